# baseline (speedup 1.0000x reference)
import functools

import jax
import jax.numpy as jnp
from jax import lax
from jax.experimental import pallas as pl
from jax.experimental.pallas import tpu as pltpu
import jax.experimental.pallas.tpu_sc as plsc

F32 = jnp.float32
I32 = jnp.int32

N = 50000
E = 800000
NP = 51200
C = 125
R = 8
RG = 2
EROWS = E // C
TS = NP // 16
WBR = 160

BLK = 2000
NB = N // BLK

_mesh = plsc.VectorSubcoreMesh(core_axis_name="c", subcore_axis_name="s")


def _fill_zero_2d(ref, nrow, ncol):
    zero16 = jnp.zeros((16,), F32)

    def body(r, _):
        for h in range(ncol // 16):
            ref[r, pl.ds(h * 16, 16)] = zero16
        return 0

    lax.fori_loop(0, nrow, body, 0)


def _zero_acc_slice(acc, wb, s, w):
    _fill_zero_2d(wb, WBR, w)
    for q in range(TS // WBR):
        pltpu.sync_copy(wb, acc.at[pl.ds(s * TS + q * WBR, WBR)])


def _lap_pass(utab, src2d, dst2d, sidx, didx, rows, acc,
              semi, semg, sems, ebase, nch, w):
    RQ = R // RG

    def idx_issue(q, p):
        pltpu.async_copy(src2d.at[pl.ds(ebase + q * R, R)], sidx.at[p],
                         semi[p])
        pltpu.async_copy(dst2d.at[pl.ds(ebase + q * R, R)], didx.at[p],
                         semi[p])

    def idx_wait(q, p):
        pltpu.make_async_copy(src2d.at[pl.ds(ebase + q * R, R)], sidx.at[p],
                              semi[p]).wait()
        pltpu.make_async_copy(dst2d.at[pl.ds(ebase + q * R, R)], didx.at[p],
                              semi[p]).wait()

    def g_issue(p, hcl, b):
        for j in range(RG):
            pltpu.async_copy(utab.at[sidx.at[p, RG * hcl + j]], rows.at[b, j],
                             semg[b])

    def g_wait(b):
        for j in range(RG):
            pltpu.make_async_copy(utab.at[sidx.at[0, 0]], rows.at[b, j],
                                  semg[b]).wait()

    def s_issue(p, hcl, b):
        for j in range(RG):
            pltpu.async_copy(rows.at[b, j], acc.at[didx.at[p, RG * hcl + j]],
                             sems[b], add=True)

    def s_wait(b):
        for j in range(RG):
            pltpu.make_async_copy(rows.at[b, j], acc.at[didx.at[0, 0]],
                                  sems[b]).wait()

    idx_issue(0, 0)
    KP = nch // 2

    def outer(k, _):
        for qq in range(2):
            q = 2 * k + qq
            idx_wait(q, qq)
            for hcl in range(RQ):
                b = hcl % 2
                first = (qq == 0 and hcl == 0)
                second = (qq == 0 and hcl == 1)
                if first or second:
                    @pl.when(k > 0)
                    def _(b=b):
                        s_wait(b)
                else:
                    s_wait(b)
                g_issue(qq, hcl, b)
                pb = 1 - b
                if hcl == 0:
                    p_prev, hcl_prev = 1 - qq, RQ - 1
                else:
                    p_prev, hcl_prev = qq, hcl - 1
                if first:
                    @pl.when(k > 0)
                    def _(pb=pb, p_prev=p_prev, hcl_prev=hcl_prev):
                        g_wait(pb)
                        s_issue(p_prev, hcl_prev, pb)
                else:
                    g_wait(pb)
                    s_issue(p_prev, hcl_prev, pb)
                if hcl == 1:
                    if qq == 0:
                        idx_issue(q + 1, 1)
                    else:
                        @pl.when(k < KP - 1)
                        def _():
                            idx_issue(q + 1, 0)
        return 0

    lax.fori_loop(0, KP, outer, 0)
    lb = (RQ - 1) % 2
    g_wait(lb)
    s_issue(1, RQ - 1, lb)
    s_wait(1 - lb)
    s_wait(lb)


def _writeback(acc, wb, d2b, disw, s, w, mk_s_dst, u_out):
    def rowmul(r, _):
        v = d2b[r, pl.ds(0, 16)]
        for h in range(w // 16):
            wb[r, pl.ds(h * 16, 16)] = wb[r, pl.ds(h * 16, 16)] * v
        return 0

    def one(q, _):
        r0 = s * TS + q * WBR
        pltpu.sync_copy(acc.at[pl.ds(r0, WBR)], wb)
        pltpu.sync_copy(disw.at[pl.ds(r0, WBR)], d2b)
        lax.fori_loop(0, WBR, rowmul, 0)
        pltpu.sync_copy(wb, mk_s_dst(r0))
        if u_out is not None:
            lax.fori_loop(0, WBR, rowmul, 0)
            pltpu.sync_copy(wb, u_out.at[pl.ds(r0, WBR)])
        return 0

    lax.fori_loop(0, TS // WBR, one, 0)


def _deg_body(src2d, p0, p1, acc, ones_r, sidx, wb, sems):
    c = lax.axis_index("c")
    s = lax.axis_index("s")
    one16 = jnp.ones((16,), F32)
    for j in range(R):
        for k in range(7):
            ones_r[j, pl.ds(k * 16, 16)] = one16
        ones_r[j, pl.ds(C - 16, 16)] = one16

    def zb(r, _):
        wb[pl.ds(r * 16, 16)] = jnp.zeros((16,), F32)
        return 0

    lax.fori_loop(0, WBR // 16, zb, 0)
    for q in range(TS // WBR):
        pltpu.sync_copy(wb, acc.at[pl.ds(s * TS + q * WBR, WBR)])
    plsc.subcore_barrier()

    ebase = c * (EROWS // 2) + s * (EROWS // 32)

    def chunk(q, _):
        row0 = ebase + q * R
        pltpu.sync_copy(src2d.at[pl.ds(row0, R)], sidx)
        hs = [pltpu.async_copy(ones_r.at[j], acc.at[sidx.at[j]],
                               sems, add=True) for j in range(R)]
        for h in hs:
            h.wait()
        return 0

    lax.fori_loop(0, (EROWS // 32) // R, chunk, 0)
    plsc.subcore_barrier()

    @pl.when(c == 0)
    def _():
        for q in range(TS // WBR):
            pltpu.sync_copy(acc.at[pl.ds(s * TS + q * WBR, WBR)], wb)
            pltpu.sync_copy(wb, p0.at[pl.ds(s * TS + q * WBR, WBR)])

    @pl.when(c == 1)
    def _():
        for q in range(TS // WBR):
            pltpu.sync_copy(acc.at[pl.ds(s * TS + q * WBR, WBR)], wb)
            pltpu.sync_copy(wb, p1.at[pl.ds(s * TS + q * WBR, WBR)])


_deg_call = pl.kernel(
    _deg_body,
    out_type=(jax.ShapeDtypeStruct((NP,), F32),
              jax.ShapeDtypeStruct((NP,), F32)),
    mesh=_mesh,
    compiler_params=pltpu.CompilerParams(use_tc_tiling_on_sc=False),
    scratch_types=[
        pltpu.VMEM_SHARED((NP,), F32),
        pltpu.VMEM((R, C), F32),
        pltpu.VMEM((R, C), I32),
        pltpu.VMEM((WBR,), F32),
        pltpu.SemaphoreType.DMA,
    ],
)


def _lap16_body(src2d, dst2d, u0, disw, s1, s2, u1,
                acc, sidx, didx, rows, wb, d2b,
                semi0, semi1, semg0, semg1, sems0, sems1):
    s = lax.axis_index("s")
    ebase = s * (EROWS // 16)
    nch = (EROWS // 16) // R
    semi, semg, sems = (semi0, semi1), (semg0, semg1), (sems0, sems1)

    _zero_acc_slice(acc, wb, s, 16)
    plsc.subcore_barrier()
    _lap_pass(u0, src2d, dst2d, sidx, didx, rows, acc, semi, semg, sems,
              ebase, nch, 16)
    plsc.subcore_barrier()
    _writeback(acc, wb, d2b, disw, s, 16,
               lambda r0: s1.at[pl.ds(r0, WBR)], u1)
    _zero_acc_slice(acc, wb, s, 16)
    plsc.subcore_barrier()
    _lap_pass(u1, src2d, dst2d, sidx, didx, rows, acc, semi, semg, sems,
              ebase, nch, 16)
    plsc.subcore_barrier()
    _writeback(acc, wb, d2b, disw, s, 16,
               lambda r0: s2.at[pl.ds(r0, WBR)], None)


_lap16_call = pl.kernel(
    _lap16_body,
    out_type=(jax.ShapeDtypeStruct((NP, 16), F32),
              jax.ShapeDtypeStruct((NP, 16), F32),
              jax.ShapeDtypeStruct((NP, 16), F32)),
    mesh=_mesh,
    compiler_params=pltpu.CompilerParams(use_tc_tiling_on_sc=False),
    scratch_types=[
        pltpu.VMEM_SHARED((NP, 16), F32),
        pltpu.VMEM((2, R, C), I32),
        pltpu.VMEM((2, R, C), I32),
        pltpu.VMEM((2, RG, C, 16), F32),
        pltpu.VMEM((WBR, 16), F32),
        pltpu.VMEM((WBR, 16), F32),
        pltpu.SemaphoreType.DMA,
        pltpu.SemaphoreType.DMA,
        pltpu.SemaphoreType.DMA,
        pltpu.SemaphoreType.DMA,
        pltpu.SemaphoreType.DMA,
        pltpu.SemaphoreType.DMA,
    ],
)


def _lap64_body(src2d, dst2d, ulo, uhi, disw, s1, s2, u1lo, u1hi,
                acc, sidx, didx, rows, wb, d2b,
                semi0, semi1, semg0, semg1, sems0, sems1):
    c = lax.axis_index("c")
    s = lax.axis_index("s")
    ebase = s * (EROWS // 16)
    nch = (EROWS // 16) // R
    semi, semg, sems = (semi0, semi1), (semg0, semg1), (sems0, sems1)

    def run(utab, u1out, ci):
        _zero_acc_slice(acc, wb, s, 32)
        plsc.subcore_barrier()
        _lap_pass(utab, src2d, dst2d, sidx, didx, rows, acc, semi, semg, sems,
                  ebase, nch, 32)
        plsc.subcore_barrier()
        _writeback(acc, wb, d2b, disw, s, 32,
                   lambda r0: s1.at[ci, pl.ds(r0, WBR)], u1out)
        _zero_acc_slice(acc, wb, s, 32)
        plsc.subcore_barrier()
        _lap_pass(u1out, src2d, dst2d, sidx, didx, rows, acc, semi, semg,
                  sems, ebase, nch, 32)
        plsc.subcore_barrier()
        _writeback(acc, wb, d2b, disw, s, 32,
                   lambda r0: s2.at[ci, pl.ds(r0, WBR)], None)

    @pl.when(c == 0)
    def _():
        run(ulo, u1lo, 0)

    @pl.when(c == 1)
    def _():
        run(uhi, u1hi, 1)


_lap64_call = pl.kernel(
    _lap64_body,
    out_type=(jax.ShapeDtypeStruct((2, NP, 32), F32),
              jax.ShapeDtypeStruct((2, NP, 32), F32),
              jax.ShapeDtypeStruct((NP, 32), F32),
              jax.ShapeDtypeStruct((NP, 32), F32)),
    mesh=_mesh,
    compiler_params=pltpu.CompilerParams(use_tc_tiling_on_sc=False),
    scratch_types=[
        pltpu.VMEM_SHARED((NP, 32), F32),
        pltpu.VMEM((2, R, C), I32),
        pltpu.VMEM((2, R, C), I32),
        pltpu.VMEM((2, RG, C, 32), F32),
        pltpu.VMEM((WBR, 32), F32),
        pltpu.VMEM((WBR, 16), F32),
        pltpu.SemaphoreType.DMA,
        pltpu.SemaphoreType.DMA,
        pltpu.SemaphoreType.DMA,
        pltpu.SemaphoreType.DMA,
        pltpu.SemaphoreType.DMA,
        pltpu.SemaphoreType.DMA,
    ],
)


def _row_spec(w):
    return pl.BlockSpec((BLK, w), lambda i: (i, 0))


def _full_spec(shape):
    nd = len(shape)
    return pl.BlockSpec(shape, lambda i: (0,) * nd)


def _prep_body(h0_r, p0_r, p1_r, u0_r, dis_r, disw_r):
    deg = p0_r[...] + p1_r[...]
    dis = jnp.where(deg > 0, deg ** -0.5, 0.0)
    u0_r[...] = h0_r[...] * dis
    dis_r[...] = dis
    disw_r[...] = jnp.broadcast_to(dis, (BLK, 16))


_prep_call = pl.pallas_call(
    _prep_body,
    grid=(NB,),
    in_specs=[_row_spec(16), _row_spec(1), _row_spec(1)],
    out_specs=[_row_spec(16), _row_spec(1), _row_spec(16)],
    out_shape=(jax.ShapeDtypeStruct((NP, 16), F32),
               jax.ShapeDtypeStruct((NP, 1), F32),
               jax.ShapeDtypeStruct((NP, 16), F32)),
)


def _dense_body(h_r, s1_r, s2_r, w0_r, w1_r, w2_r, b_r, a_r,
                z_r, st_r, sacc):
    i = pl.program_id(0)
    z = jnp.dot(h_r[...], w0_r[...], preferred_element_type=F32, precision=lax.Precision.HIGHEST)
    z = z + jnp.dot(s1_r[...], w1_r[...], preferred_element_type=F32, precision=lax.Precision.HIGHEST)
    z = z + jnp.dot(s2_r[...], w2_r[...], preferred_element_type=F32, precision=lax.Precision.HIGHEST)
    z = z + b_r[...]
    a = a_r[0, 0]
    z = jnp.where(z >= 0, z, a * z)
    z_r[...] = z

    @pl.when(i == 0)
    def _():
        sacc[...] = jnp.zeros_like(sacc)

    sacc[...] += jnp.concatenate(
        [jnp.sum(z, axis=0, keepdims=True),
         jnp.sum(z * z, axis=0, keepdims=True)], axis=0)

    @pl.when(i == NB - 1)
    def _():
        st_r[...] = sacc[...]


_dense1_call = pl.pallas_call(
    _dense_body,
    grid=(NB,),
    in_specs=[_row_spec(16), _row_spec(16), _row_spec(16),
              _full_spec((16, 64)), _full_spec((16, 64)),
              _full_spec((16, 64)), _full_spec((1, 64)),
              _full_spec((1, 1))],
    out_specs=[_row_spec(64), _full_spec((2, 64))],
    out_shape=(jax.ShapeDtypeStruct((NP, 64), F32),
               jax.ShapeDtypeStruct((2, 64), F32)),
    scratch_shapes=[pltpu.VMEM((2, 64), F32)],
)


def _dense_split_body(h_r, s1lo_r, s1hi_r, s2lo_r, s2hi_r,
                      w0_r, w1a_r, w1b_r, w2a_r, w2b_r, b_r, a_r,
                      z_r, st_r, sacc):
    i = pl.program_id(0)
    z = jnp.dot(h_r[...], w0_r[...], preferred_element_type=F32, precision=lax.Precision.HIGHEST)
    z = z + jnp.dot(s1lo_r[0], w1a_r[...], preferred_element_type=F32, precision=lax.Precision.HIGHEST)
    z = z + jnp.dot(s1hi_r[0], w1b_r[...], preferred_element_type=F32, precision=lax.Precision.HIGHEST)
    z = z + jnp.dot(s2lo_r[0], w2a_r[...], preferred_element_type=F32, precision=lax.Precision.HIGHEST)
    z = z + jnp.dot(s2hi_r[0], w2b_r[...], preferred_element_type=F32, precision=lax.Precision.HIGHEST)
    z = z + b_r[...]
    a = a_r[0, 0]
    z = jnp.where(z >= 0, z, a * z)
    z_r[...] = z

    @pl.when(i == 0)
    def _():
        sacc[...] = jnp.zeros_like(sacc)

    sacc[...] += jnp.concatenate(
        [jnp.sum(z, axis=0, keepdims=True),
         jnp.sum(z * z, axis=0, keepdims=True)], axis=0)

    @pl.when(i == NB - 1)
    def _():
        st_r[...] = sacc[...]


def _make_dense_split(dout):
    half = pl.BlockSpec((1, BLK, 32), lambda i: (0, i, 0))
    half_hi = pl.BlockSpec((1, BLK, 32), lambda i: (1, i, 0))
    return pl.pallas_call(
        _dense_split_body,
        grid=(NB,),
        in_specs=[_row_spec(64), half, half_hi, half, half_hi,
                  _full_spec((64, dout)),
                  _full_spec((32, dout)), _full_spec((32, dout)),
                  _full_spec((32, dout)), _full_spec((32, dout)),
                  _full_spec((1, dout)), _full_spec((1, 1))],
        out_specs=[_row_spec(dout), _full_spec((2, dout))],
        out_shape=(jax.ShapeDtypeStruct((NP, dout), F32),
                   jax.ShapeDtypeStruct((2, dout), F32)),
        scratch_shapes=[pltpu.VMEM((2, dout), F32)],
    )


_dense_mid_call = _make_dense_split(64)
_dense4_call = _make_dense_split(32)


def _norm_split_body(z_r, st_r, g_r, be_r, dis_r, h_r, ulo_r, uhi_r):
    st = st_r[...]
    mu = st[0:1, :] * (1.0 / N)
    var = st[1:2, :] * (1.0 / N) - mu * mu
    rstd = lax.rsqrt(var + 1e-5)
    h = (z_r[...] - mu) * rstd * g_r[...] + be_r[...]
    h_r[...] = h
    u = h * dis_r[...]
    ulo_r[...] = u[:, :32]
    uhi_r[...] = u[:, 32:]


_norm_split_call = pl.pallas_call(
    _norm_split_body,
    grid=(NB,),
    in_specs=[_row_spec(64), _full_spec((2, 64)), _full_spec((1, 64)),
              _full_spec((1, 64)), _row_spec(1)],
    out_specs=[_row_spec(64), _row_spec(32), _row_spec(32)],
    out_shape=(jax.ShapeDtypeStruct((N, 64), F32),
               jax.ShapeDtypeStruct((NP, 32), F32),
               jax.ShapeDtypeStruct((NP, 32), F32)),
)


def _norm_final_body(z_r, st_r, g_r, be_r, h_r):
    st = st_r[...]
    mu = st[0:1, :] * (1.0 / N)
    var = st[1:2, :] * (1.0 / N) - mu * mu
    rstd = lax.rsqrt(var + 1e-5)
    h_r[...] = (z_r[...] - mu) * rstd * g_r[...] + be_r[...]


_norm_final_call = pl.pallas_call(
    _norm_final_body,
    grid=(NB,),
    in_specs=[_row_spec(32), _full_spec((2, 32)), _full_spec((1, 32)),
              _full_spec((1, 32))],
    out_specs=_row_spec(32),
    out_shape=jax.ShapeDtypeStruct((N, 32), F32),
)


def _fold_weights(W, din_pad=None):
    w0, w1, w2 = W[0], W[1], W[2]
    wc0 = w0 - w2
    wc1 = -w1
    wc2 = 2.0 * w2
    if din_pad is not None:
        pad = ((0, din_pad - w0.shape[0]), (0, 0))
        wc0 = jnp.pad(wc0, pad)
        wc1 = jnp.pad(wc1, pad)
        wc2 = jnp.pad(wc2, pad)
        return wc0, wc1, wc2
    return wc0, wc1[:32], wc1[32:], wc2[:32], wc2[32:]


def kernel(x, pos, norm, edge_index,
           W1, b1, a1, g1, be1,
           W2, b2, a2, g2, be2,
           W3, b3, a3, g3, be3,
           W4, b4, a4, g4, be4):
    src2d = edge_index[0].reshape(EROWS, C)
    dst2d = edge_index[1].reshape(EROWS, C)

    p0, p1 = _deg_call(src2d)

    h0 = jnp.concatenate([x, pos, norm, jnp.zeros((N, 7), F32)], axis=1)
    u0, dis_c, disw = _prep_call(h0, p0.reshape(NP, 1), p1.reshape(NP, 1))

    s1, s2, _u1 = _lap16_call(src2d, dst2d, u0, disw)
    wc0, wc1, wc2 = _fold_weights(W1, 16)
    z, st = _dense1_call(h0, s1, s2,
                         wc0, wc1, wc2, b1.reshape(1, 64),
                         a1.reshape(1, 1))
    h, ulo, uhi = _norm_split_call(z, st, g1.reshape(1, 64),
                                   be1.reshape(1, 64), dis_c)

    for (W, b, a, g, be) in ((W2, b2, a2, g2, be2), (W3, b3, a3, g3, be3)):
        s1, s2, _ulo1, _uhi1 = _lap64_call(src2d, dst2d, ulo, uhi, disw)
        wc0, w1a, w1b, w2a, w2b = _fold_weights(W)
        z, st = _dense_mid_call(h, s1, s1, s2, s2,
                                wc0, w1a, w1b, w2a, w2b, b.reshape(1, 64),
                                a.reshape(1, 1))
        h, ulo, uhi = _norm_split_call(z, st, g.reshape(1, 64),
                                       be.reshape(1, 64), dis_c)

    s1, s2, _ulo1, _uhi1 = _lap64_call(src2d, dst2d, ulo, uhi, disw)
    wc0, w1a, w1b, w2a, w2b = _fold_weights(W4)
    z, st = _dense4_call(h, s1, s1, s2, s2,
                         wc0, w1a, w1b, w2a, w2b, b4.reshape(1, 32),
                         a4.reshape(1, 1))
    return _norm_final_call(z, st, g4.reshape(1, 32), be4.reshape(1, 32))


# speedup vs baseline: 1.1081x; 1.1081x over previous
import functools

import jax
import jax.numpy as jnp
from jax import lax
from jax.experimental import pallas as pl
from jax.experimental.pallas import tpu as pltpu
import jax.experimental.pallas.tpu_sc as plsc

F32 = jnp.float32
I32 = jnp.int32

N = 50000
E = 800000
NP = 51200
C = 125
R = 8
RG = 2
EROWS = E // C
TS = NP // 16
WBR = 160

BLK = 2000
NB = N // BLK

_mesh = plsc.VectorSubcoreMesh(core_axis_name="c", subcore_axis_name="s")


def _fill_zero_2d(ref, nrow, ncol):
    zero16 = jnp.zeros((16,), F32)

    def body(r, _):
        for h in range(ncol // 16):
            ref[r, pl.ds(h * 16, 16)] = zero16
        return 0

    lax.fori_loop(0, nrow, body, 0)


def _zero_acc_slice(acc, wb, s, w):
    _fill_zero_2d(wb, WBR, w)
    for q in range(TS // WBR):
        pltpu.sync_copy(wb, acc.at[pl.ds(s * TS + q * WBR, WBR)])


def _lap_pass(utab, src2d, dst2d, sidx, didx, rows, acc,
              semi, semg, sems, ebase, nch, w):
    RQ = R // RG

    def idx_issue(q, p):
        pltpu.async_copy(src2d.at[pl.ds(ebase + q * R, R)], sidx.at[p],
                         semi[p])
        pltpu.async_copy(dst2d.at[pl.ds(ebase + q * R, R)], didx.at[p],
                         semi[p])

    def idx_wait(q, p):
        pltpu.make_async_copy(src2d.at[pl.ds(ebase + q * R, R)], sidx.at[p],
                              semi[p]).wait()
        pltpu.make_async_copy(dst2d.at[pl.ds(ebase + q * R, R)], didx.at[p],
                              semi[p]).wait()

    def g_issue(p, hcl, b):
        for j in range(RG):
            pltpu.async_copy(utab.at[sidx.at[p, RG * hcl + j]], rows.at[b, j],
                             semg[b])

    def g_wait(b):
        for j in range(RG):
            pltpu.make_async_copy(utab.at[sidx.at[0, 0]], rows.at[b, j],
                                  semg[b]).wait()

    def s_issue(p, hcl, b):
        for j in range(RG):
            pltpu.async_copy(rows.at[b, j], acc.at[didx.at[p, RG * hcl + j]],
                             sems[b], add=True)

    def s_wait(b):
        for j in range(RG):
            pltpu.make_async_copy(rows.at[b, j], acc.at[didx.at[0, 0]],
                                  sems[b]).wait()

    idx_issue(0, 0)
    KP = nch // 2

    def outer(k, _):
        for qq in range(2):
            q = 2 * k + qq
            idx_wait(q, qq)
            for hcl in range(RQ):
                b = hcl % 2
                first = (qq == 0 and hcl == 0)
                second = (qq == 0 and hcl == 1)
                if first or second:
                    @pl.when(k > 0)
                    def _(b=b):
                        s_wait(b)
                else:
                    s_wait(b)
                g_issue(qq, hcl, b)
                pb = 1 - b
                if hcl == 0:
                    p_prev, hcl_prev = 1 - qq, RQ - 1
                else:
                    p_prev, hcl_prev = qq, hcl - 1
                if first:
                    @pl.when(k > 0)
                    def _(pb=pb, p_prev=p_prev, hcl_prev=hcl_prev):
                        g_wait(pb)
                        s_issue(p_prev, hcl_prev, pb)
                else:
                    g_wait(pb)
                    s_issue(p_prev, hcl_prev, pb)
                if hcl == 1:
                    if qq == 0:
                        idx_issue(q + 1, 1)
                    else:
                        @pl.when(k < KP - 1)
                        def _():
                            idx_issue(q + 1, 0)
        return 0

    lax.fori_loop(0, KP, outer, 0)
    lb = (RQ - 1) % 2
    g_wait(lb)
    s_issue(1, RQ - 1, lb)
    s_wait(1 - lb)
    s_wait(lb)


def _writeback(acc, wb, d2b, disw, s, w, mk_s_dst, u_out):
    def rowmul(r4, _):
        for dr in range(4):
            r = r4 * 4 + dr
            v = d2b[r, pl.ds(0, 16)]
            for h in range(w // 16):
                wb[r, pl.ds(h * 16, 16)] = wb[r, pl.ds(h * 16, 16)] * v
        return 0

    def one(q, _):
        r0 = s * TS + q * WBR
        pltpu.sync_copy(acc.at[pl.ds(r0, WBR)], wb)
        pltpu.sync_copy(disw.at[pl.ds(r0, WBR)], d2b)
        lax.fori_loop(0, WBR // 4, rowmul, 0)
        pltpu.sync_copy(wb, mk_s_dst(r0))
        if u_out is not None:
            lax.fori_loop(0, WBR // 4, rowmul, 0)
            pltpu.sync_copy(wb, u_out.at[pl.ds(r0, WBR)])
        return 0

    lax.fori_loop(0, TS // WBR, one, 0)


def _deg_body(src2d, p0, p1, acc, ones_r, sidx, wb, sems):
    c = lax.axis_index("c")
    s = lax.axis_index("s")
    one16 = jnp.ones((16,), F32)
    for j in range(R):
        for k in range(7):
            ones_r[j, pl.ds(k * 16, 16)] = one16
        ones_r[j, pl.ds(C - 16, 16)] = one16

    def zb(r, _):
        wb[pl.ds(r * 16, 16)] = jnp.zeros((16,), F32)
        return 0

    lax.fori_loop(0, WBR // 16, zb, 0)
    for q in range(TS // WBR):
        pltpu.sync_copy(wb, acc.at[pl.ds(s * TS + q * WBR, WBR)])
    plsc.subcore_barrier()

    ebase = c * (EROWS // 2) + s * (EROWS // 32)

    def chunk(q, _):
        row0 = ebase + q * R
        pltpu.sync_copy(src2d.at[pl.ds(row0, R)], sidx)
        hs = [pltpu.async_copy(ones_r.at[j], acc.at[sidx.at[j]],
                               sems, add=True) for j in range(R)]
        for h in hs:
            h.wait()
        return 0

    lax.fori_loop(0, (EROWS // 32) // R, chunk, 0)
    plsc.subcore_barrier()

    @pl.when(c == 0)
    def _():
        for q in range(TS // WBR):
            pltpu.sync_copy(acc.at[pl.ds(s * TS + q * WBR, WBR)], wb)
            pltpu.sync_copy(wb, p0.at[pl.ds(s * TS + q * WBR, WBR)])

    @pl.when(c == 1)
    def _():
        for q in range(TS // WBR):
            pltpu.sync_copy(acc.at[pl.ds(s * TS + q * WBR, WBR)], wb)
            pltpu.sync_copy(wb, p1.at[pl.ds(s * TS + q * WBR, WBR)])


_deg_call = pl.kernel(
    _deg_body,
    out_type=(jax.ShapeDtypeStruct((NP,), F32),
              jax.ShapeDtypeStruct((NP,), F32)),
    mesh=_mesh,
    compiler_params=pltpu.CompilerParams(use_tc_tiling_on_sc=False),
    scratch_types=[
        pltpu.VMEM_SHARED((NP,), F32),
        pltpu.VMEM((R, C), F32),
        pltpu.VMEM((R, C), I32),
        pltpu.VMEM((WBR,), F32),
        pltpu.SemaphoreType.DMA,
    ],
)


def _lap16_body(src2d, dst2d, u0, disw, s1, s2, u1,
                acc, sidx, didx, rows, wb, d2b,
                semi0, semi1, semg0, semg1, sems0, sems1):
    s = lax.axis_index("s")
    ebase = s * (EROWS // 16)
    nch = (EROWS // 16) // R
    semi, semg, sems = (semi0, semi1), (semg0, semg1), (sems0, sems1)

    _zero_acc_slice(acc, wb, s, 16)
    plsc.subcore_barrier()
    _lap_pass(u0, src2d, dst2d, sidx, didx, rows, acc, semi, semg, sems,
              ebase, nch, 16)
    plsc.subcore_barrier()
    _writeback(acc, wb, d2b, disw, s, 16,
               lambda r0: s1.at[pl.ds(r0, WBR)], u1)
    _zero_acc_slice(acc, wb, s, 16)
    plsc.subcore_barrier()
    _lap_pass(u1, src2d, dst2d, sidx, didx, rows, acc, semi, semg, sems,
              ebase, nch, 16)
    plsc.subcore_barrier()
    _writeback(acc, wb, d2b, disw, s, 16,
               lambda r0: s2.at[pl.ds(r0, WBR)], None)


_lap16_call = pl.kernel(
    _lap16_body,
    out_type=(jax.ShapeDtypeStruct((NP, 16), F32),
              jax.ShapeDtypeStruct((NP, 16), F32),
              jax.ShapeDtypeStruct((NP, 16), F32)),
    mesh=_mesh,
    compiler_params=pltpu.CompilerParams(use_tc_tiling_on_sc=False),
    scratch_types=[
        pltpu.VMEM_SHARED((NP, 16), F32),
        pltpu.VMEM((2, R, C), I32),
        pltpu.VMEM((2, R, C), I32),
        pltpu.VMEM((2, RG, C, 16), F32),
        pltpu.VMEM((WBR, 16), F32),
        pltpu.VMEM((WBR, 16), F32),
        pltpu.SemaphoreType.DMA,
        pltpu.SemaphoreType.DMA,
        pltpu.SemaphoreType.DMA,
        pltpu.SemaphoreType.DMA,
        pltpu.SemaphoreType.DMA,
        pltpu.SemaphoreType.DMA,
    ],
)


def _lap64_body(src2d, dst2d, ulo, uhi, disw, s1, s2, u1lo, u1hi,
                acc, sidx, didx, rows, wb, d2b,
                semi0, semi1, semg0, semg1, sems0, sems1):
    c = lax.axis_index("c")
    s = lax.axis_index("s")
    ebase = s * (EROWS // 16)
    nch = (EROWS // 16) // R
    semi, semg, sems = (semi0, semi1), (semg0, semg1), (sems0, sems1)

    def run(utab, u1out, ci):
        _zero_acc_slice(acc, wb, s, 32)
        plsc.subcore_barrier()
        _lap_pass(utab, src2d, dst2d, sidx, didx, rows, acc, semi, semg, sems,
                  ebase, nch, 32)
        plsc.subcore_barrier()
        _writeback(acc, wb, d2b, disw, s, 32,
                   lambda r0: s1.at[pl.ds(r0, WBR), pl.ds(ci * 32, 32)],
                   u1out)
        _zero_acc_slice(acc, wb, s, 32)
        plsc.subcore_barrier()
        _lap_pass(u1out, src2d, dst2d, sidx, didx, rows, acc, semi, semg,
                  sems, ebase, nch, 32)
        plsc.subcore_barrier()
        _writeback(acc, wb, d2b, disw, s, 32,
                   lambda r0: s2.at[pl.ds(r0, WBR), pl.ds(ci * 32, 32)],
                   None)

    @pl.when(c == 0)
    def _():
        run(ulo, u1lo, 0)

    @pl.when(c == 1)
    def _():
        run(uhi, u1hi, 1)


_lap64_call = pl.kernel(
    _lap64_body,
    out_type=(jax.ShapeDtypeStruct((NP, 64), F32),
              jax.ShapeDtypeStruct((NP, 64), F32),
              jax.ShapeDtypeStruct((NP, 32), F32),
              jax.ShapeDtypeStruct((NP, 32), F32)),
    mesh=_mesh,
    compiler_params=pltpu.CompilerParams(use_tc_tiling_on_sc=False),
    scratch_types=[
        pltpu.VMEM_SHARED((NP, 32), F32),
        pltpu.VMEM((2, R, C), I32),
        pltpu.VMEM((2, R, C), I32),
        pltpu.VMEM((2, RG, C, 32), F32),
        pltpu.VMEM((WBR, 32), F32),
        pltpu.VMEM((WBR, 16), F32),
        pltpu.SemaphoreType.DMA,
        pltpu.SemaphoreType.DMA,
        pltpu.SemaphoreType.DMA,
        pltpu.SemaphoreType.DMA,
        pltpu.SemaphoreType.DMA,
        pltpu.SemaphoreType.DMA,
    ],
)


def _row_spec(w):
    return pl.BlockSpec((BLK, w), lambda i: (i, 0))


def _full_spec(shape):
    nd = len(shape)
    return pl.BlockSpec(shape, lambda i: (0,) * nd)


def _prep_body(h0_r, p0_r, p1_r, u0_r, dis_r, disw_r):
    deg = p0_r[...] + p1_r[...]
    dis = jnp.where(deg > 0, deg ** -0.5, 0.0)
    u0_r[...] = h0_r[...] * dis
    dis_r[...] = dis
    disw_r[...] = jnp.broadcast_to(dis, (BLK, 16))


_prep_call = pl.pallas_call(
    _prep_body,
    grid=(NB,),
    in_specs=[_row_spec(16), _row_spec(1), _row_spec(1)],
    out_specs=[_row_spec(16), _row_spec(1), _row_spec(16)],
    out_shape=(jax.ShapeDtypeStruct((NP, 16), F32),
               jax.ShapeDtypeStruct((NP, 1), F32),
               jax.ShapeDtypeStruct((NP, 16), F32)),
)


def _dense_body(h_r, s1_r, s2_r, w0_r, w1_r, w2_r, b_r, a_r,
                z_r, st_r, sacc):
    i = pl.program_id(0)
    z = jnp.dot(h_r[...], w0_r[...], preferred_element_type=F32, precision=lax.Precision.HIGHEST)
    z = z + jnp.dot(s1_r[...], w1_r[...], preferred_element_type=F32, precision=lax.Precision.HIGHEST)
    z = z + jnp.dot(s2_r[...], w2_r[...], preferred_element_type=F32, precision=lax.Precision.HIGHEST)
    z = z + b_r[...]
    a = a_r[0, 0]
    z = jnp.where(z >= 0, z, a * z)
    z_r[...] = z

    @pl.when(i == 0)
    def _():
        sacc[...] = jnp.zeros_like(sacc)

    sacc[...] += jnp.concatenate(
        [jnp.sum(z, axis=0, keepdims=True),
         jnp.sum(z * z, axis=0, keepdims=True)], axis=0)

    @pl.when(i == NB - 1)
    def _():
        st_r[...] = sacc[...]


def _make_dense(din, dout):
    return pl.pallas_call(
        _dense_body,
        grid=(NB,),
        in_specs=[_row_spec(din), _row_spec(din), _row_spec(din),
                  _full_spec((din, dout)), _full_spec((din, dout)),
                  _full_spec((din, dout)), _full_spec((1, dout)),
                  _full_spec((1, 1))],
        out_specs=[_row_spec(dout), _full_spec((2, dout))],
        out_shape=(jax.ShapeDtypeStruct((NP, dout), F32),
                   jax.ShapeDtypeStruct((2, dout), F32)),
        scratch_shapes=[pltpu.VMEM((2, dout), F32)],
    )


_dense1_call = _make_dense(16, 64)
_dense_mid_call = _make_dense(64, 64)
_dense4_call = _make_dense(64, 32)


def _norm_split_body(z_r, st_r, g_r, be_r, dis_r, h_r, ulo_r, uhi_r):
    st = st_r[...]
    mu = st[0:1, :] * (1.0 / N)
    var = st[1:2, :] * (1.0 / N) - mu * mu
    rstd = lax.rsqrt(var + 1e-5)
    h = (z_r[...] - mu) * rstd * g_r[...] + be_r[...]
    h_r[...] = h
    u = h * dis_r[...]
    ulo_r[...] = u[:, :32]
    uhi_r[...] = u[:, 32:]


_norm_split_call = pl.pallas_call(
    _norm_split_body,
    grid=(NB,),
    in_specs=[_row_spec(64), _full_spec((2, 64)), _full_spec((1, 64)),
              _full_spec((1, 64)), _row_spec(1)],
    out_specs=[_row_spec(64), _row_spec(32), _row_spec(32)],
    out_shape=(jax.ShapeDtypeStruct((N, 64), F32),
               jax.ShapeDtypeStruct((NP, 32), F32),
               jax.ShapeDtypeStruct((NP, 32), F32)),
)


def _norm_final_body(z_r, st_r, g_r, be_r, h_r):
    st = st_r[...]
    mu = st[0:1, :] * (1.0 / N)
    var = st[1:2, :] * (1.0 / N) - mu * mu
    rstd = lax.rsqrt(var + 1e-5)
    h_r[...] = (z_r[...] - mu) * rstd * g_r[...] + be_r[...]


_norm_final_call = pl.pallas_call(
    _norm_final_body,
    grid=(NB,),
    in_specs=[_row_spec(32), _full_spec((2, 32)), _full_spec((1, 32)),
              _full_spec((1, 32))],
    out_specs=_row_spec(32),
    out_shape=jax.ShapeDtypeStruct((N, 32), F32),
)


def _fold_weights(W, din_pad=None):
    w0, w1, w2 = W[0], W[1], W[2]
    wc0 = w0 - w2
    wc1 = -w1
    wc2 = 2.0 * w2
    if din_pad is not None:
        pad = ((0, din_pad - w0.shape[0]), (0, 0))
        wc0 = jnp.pad(wc0, pad)
        wc1 = jnp.pad(wc1, pad)
        wc2 = jnp.pad(wc2, pad)
    return wc0, wc1, wc2


def kernel(x, pos, norm, edge_index,
           W1, b1, a1, g1, be1,
           W2, b2, a2, g2, be2,
           W3, b3, a3, g3, be3,
           W4, b4, a4, g4, be4):
    src2d = edge_index[0].reshape(EROWS, C)
    dst2d = edge_index[1].reshape(EROWS, C)

    p0, p1 = _deg_call(src2d)

    h0 = jnp.concatenate([x, pos, norm, jnp.zeros((N, 7), F32)], axis=1)
    u0, dis_c, disw = _prep_call(h0, p0.reshape(NP, 1), p1.reshape(NP, 1))

    s1, s2, _u1 = _lap16_call(src2d, dst2d, u0, disw)
    wc0, wc1, wc2 = _fold_weights(W1, 16)
    z, st = _dense1_call(h0, s1, s2,
                         wc0, wc1, wc2, b1.reshape(1, 64),
                         a1.reshape(1, 1))
    h, ulo, uhi = _norm_split_call(z, st, g1.reshape(1, 64),
                                   be1.reshape(1, 64), dis_c)

    for (W, b, a, g, be) in ((W2, b2, a2, g2, be2), (W3, b3, a3, g3, be3)):
        s1, s2, _ulo1, _uhi1 = _lap64_call(src2d, dst2d, ulo, uhi, disw)
        wc0, wc1, wc2 = _fold_weights(W)
        z, st = _dense_mid_call(h, s1, s2,
                                wc0, wc1, wc2, b.reshape(1, 64),
                                a.reshape(1, 1))
        h, ulo, uhi = _norm_split_call(z, st, g.reshape(1, 64),
                                       be.reshape(1, 64), dis_c)

    s1, s2, _ulo1, _uhi1 = _lap64_call(src2d, dst2d, ulo, uhi, disw)
    wc0, wc1, wc2 = _fold_weights(W4)
    z, st = _dense4_call(h, s1, s2,
                         wc0, wc1, wc2, b4.reshape(1, 32),
                         a4.reshape(1, 1))
    return _norm_final_call(z, st, g4.reshape(1, 32), be4.reshape(1, 32))
